# baseline (speedup 1.0000x reference)
import math

import jax
import jax.numpy as jnp
import numpy as np
from jax.experimental import pallas as pl
from jax.experimental.pallas import tpu as pltpu

_N = 512
_V = 4
_F = 64
_H = 64
_BR = 32
_L = _BR * _N

_RS132 = 1.0 / math.sqrt(_V + 2 * _F)
_RS64 = 1.0 / math.sqrt(64.0)
_RS128 = 1.0 / math.sqrt(128.0)
_INV_DEG = 1.0 / (_N - 1)


def _sigmoid(x):
    return 0.5 * jnp.tanh(0.5 * x) + 0.5


def _silu(x):
    y = 0.5 * x
    return y * (jnp.tanh(y) + 1.0)


def _egcl_body(nv_ref, nf_ref, nv_full_ref, nf_full_ref, R_ref, RT_ref,
               We1_ref, We2_ref, Wx1_ref, Wx2_ref, Winf_ref, Wlin_ref,
               blin_ref, Wh1_ref, Wh2_ref, Wout_ref,
               vout_ref, fout_ref,
               nvT_t_ref, A_sT_t_ref,
               We1vT_ref, We1rT_ref, We2T_ref, Wx1T_ref, Wx2T_ref,
               WinfT_ref, WlinT_ref, PE_ref, RD_ref):
    i = pl.program_id(0)
    f32 = jnp.float32

    bf16 = jnp.bfloat16

    @pl.when(i == 0)
    def _():
        We1vT_ref[:, :] = (We1_ref[:_V, :].T * _RS132).astype(bf16)
        We1rT_ref[:, :] = We1_ref[_V + _F:, :].T * _RS132
        We2T_ref[:, :] = (We2_ref[:, :].T * _RS64).astype(bf16)
        Wx1T_ref[:, :] = (Wx1_ref[:, :].T * _RS64).astype(bf16)
        Wx2T_ref[:, :] = (Wx2_ref[:, :].T * _RS64).astype(bf16)
        WinfT_ref[:, :] = (Winf_ref[:, :].T * _RS64).astype(bf16)
        WlinT_ref[:, :] = Wlin_ref[:, :].T.astype(bf16)

        nvT = nv_full_ref[:, :].T
        A_s = jnp.dot(nf_full_ref[:, :], We1_ref[_V:_V + _F, :],
                      preferred_element_type=f32) * _RS132
        A_sT = A_s.T
        nvT_t_ref[:, :] = jnp.concatenate([nvT] * _BR, axis=1)
        A_sT_t_ref[:, :] = jnp.concatenate([A_sT] * _BR, axis=1)
        PE_ref[12:16, :] = jnp.zeros((4, _BR), f32)
        RD_ref[12:16, :] = jnp.zeros((4, _L), bf16)

    r0 = i * _BR
    lane = jax.lax.broadcasted_iota(jnp.int32, (1, _L), 1)
    s_id = lane % _N
    r_id = r0 + lane // _N
    mask = (s_id != r_id).astype(f32)

    R = R_ref[:, :]
    RT = RT_ref[:, :]

    nf_blk = nf_ref[:, :]
    nv_blk = nv_ref[:, :]
    PE_ref[0:12, :] = nv_blk.T
    PE_ref[16:80, :] = jnp.dot(We1rT_ref[:, :], nf_blk.T,
                               preferred_element_type=f32)
    exp = jnp.dot(PE_ref[:, :], R, preferred_element_type=f32)
    nv_r_exp = exp[0:12, :]
    A_r_exp = exp[16:80, :]

    diff = nv_r_exp - nvT_t_ref[:, :]
    row = jax.lax.broadcasted_iota(jnp.int32, (_V, 12), 0)
    col = jax.lax.broadcasted_iota(jnp.int32, (_V, 12), 1)
    sel4 = (col // 3 == row).astype(f32)
    n2 = jnp.dot(sel4, diff * diff,
                 preferred_element_type=f32)
    length = jnp.sqrt(jnp.maximum(n2, 1e-20))

    h1 = jnp.dot(We1vT_ref[:, :], n2.astype(bf16),
                 preferred_element_type=f32)
    h1 = _silu(h1 + A_sT_t_ref[:, :] + A_r_exp)
    m = _silu(jnp.dot(We2T_ref[:, :], h1.astype(bf16),
                      preferred_element_type=f32))
    mb = m.astype(bf16)
    p = _silu(jnp.dot(Wx1T_ref[:, :], mb,
                      preferred_element_type=f32))
    p = _silu(jnp.dot(Wx2T_ref[:, :], p.astype(bf16),
                      preferred_element_type=f32))
    phi = jnp.dot(WlinT_ref[:, :], p.astype(bf16),
                  preferred_element_type=f32) + blin_ref[:, :]

    g = phi / (1.0 + length) * mask
    g12 = jnp.dot(sel4.T, g, preferred_element_type=f32)
    e = _sigmoid(jnp.dot(WinfT_ref[:, :], mb,
                         preferred_element_type=f32))
    RD_ref[0:12, :] = (g12 * diff).astype(bf16)
    RD_ref[16:80, :] = (m * (e * mask)).astype(bf16)
    red = jnp.dot(RD_ref[:, :], RT, preferred_element_type=f32)
    shifts = red[0:12, :]
    m_i = red[16:80, :]

    m_i_n = m_i.T
    hh = jnp.dot(m_i_n, Wh1_ref[:_H, :], preferred_element_type=f32)
    hh = hh + jnp.dot(nf_blk, Wh1_ref[_H:, :], preferred_element_type=f32)
    hh = _silu(hh * _RS128)
    hh = _silu(jnp.dot(hh, Wh2_ref[:, :],
                       preferred_element_type=f32) * _RS64)
    fout_ref[:, :] = (jnp.dot(hh, Wout_ref[:, :],
                              preferred_element_type=f32) * _RS64 + nf_blk)

    vout_ref[:, :] = nv_blk + shifts.T * _INV_DEG


@jax.jit
def kernel(node_vectors, node_features, We1, We2, Wx1, Wx2, Winf, Wlin,
           blin, Wh1, Wh2, Wout):
    f32 = jnp.float32
    nv = node_vectors.reshape(_N, _V * 3).astype(f32)
    nf = node_features.astype(f32)

    seg = np.arange(_L) // _N
    R = jnp.asarray(seg[None, :] == np.arange(_BR)[:, None], dtype=f32)
    RT = jnp.asarray(seg[:, None] == np.arange(_BR)[None, :],
                     dtype=jnp.bfloat16)

    grid = (_N // _BR,)

    def full(shape):
        nd = len(shape)
        return pl.BlockSpec(shape, lambda i: (0,) * nd)

    out_shape = [
        jax.ShapeDtypeStruct((_N, 12), f32),
        jax.ShapeDtypeStruct((_N, _F), f32),
    ]
    out_specs = [
        pl.BlockSpec((_BR, 12), lambda i: (i, 0)),
        pl.BlockSpec((_BR, _F), lambda i: (i, 0)),
    ]
    in_arrays = [
        nv, nf, nv, nf, R, RT,
        We1, We2, Wx1, Wx2, Winf, Wlin, blin.reshape(_V, 1),
        Wh1, Wh2, Wout,
    ]
    in_specs = [full(a.shape) for a in in_arrays]
    in_specs[0] = pl.BlockSpec((_BR, 12), lambda i: (i, 0))
    in_specs[1] = pl.BlockSpec((_BR, _F), lambda i: (i, 0))

    vout, fout = pl.pallas_call(
        _egcl_body,
        grid=grid,
        in_specs=in_specs,
        out_specs=out_specs,
        out_shape=out_shape,
        scratch_shapes=[
            pltpu.VMEM((12, _L), f32),
            pltpu.VMEM((64, _L), f32),
            pltpu.VMEM((64, _V), jnp.bfloat16),
            pltpu.VMEM((64, 64), f32),
            pltpu.VMEM((64, 64), jnp.bfloat16),
            pltpu.VMEM((64, 64), jnp.bfloat16),
            pltpu.VMEM((64, 64), jnp.bfloat16),
            pltpu.VMEM((1, 64), jnp.bfloat16),
            pltpu.VMEM((_V, 64), jnp.bfloat16),
            pltpu.VMEM((80, _BR), f32),
            pltpu.VMEM((80, _L), jnp.bfloat16),
        ],
    )(*in_arrays)

    return vout.reshape(_N, _V, 3), fout


# speedup vs baseline: 1.0215x; 1.0215x over previous
import math

import jax
import jax.numpy as jnp
import numpy as np
from jax.experimental import pallas as pl
from jax.experimental.pallas import tpu as pltpu

_N = 512
_V = 4
_F = 64
_H = 64
_BR = 32
_L = _BR * _N

_RS132 = 1.0 / math.sqrt(_V + 2 * _F)
_RS64 = 1.0 / math.sqrt(64.0)
_RS128 = 1.0 / math.sqrt(128.0)
_INV_DEG = 1.0 / (_N - 1)


def _sigmoid(x):
    return 0.5 * jnp.tanh(0.5 * x) + 0.5


def _silu(x):
    y = 0.5 * x
    return y * (jnp.tanh(y) + 1.0)


def _egcl_body(nv_ref, nf_ref, nv_full_ref, nf_full_ref, R_ref, RT_ref,
               We1_ref, We2_ref, Wx1_ref, Wx2_ref, Winf_ref, Wlin_ref,
               blin_ref, Wh1_ref, Wh2_ref, Wout_ref,
               vout_ref, fout_ref,
               nvT_t_ref, A_sT_t_ref,
               P1_ref, We1rT_ref, We2T_ref, Wx1T_ref, Wx2T_ref,
               WinfT_ref, WlinT_ref, PE_ref, RD_ref):
    i = pl.program_id(0)
    f32 = jnp.float32

    @pl.when(i == 0)
    def _():
        We1rT_ref[:, :] = We1_ref[_V + _F:, :].T * _RS132
        We2T_ref[:, :] = We2_ref[:, :].T * _RS64
        Wx1T_ref[:, :] = Wx1_ref[:, :].T * _RS64
        Wx2T_ref[:, :] = Wx2_ref[:, :].T * _RS64
        WinfT_ref[:, :] = Winf_ref[:, :].T * _RS64
        WlinT_ref[:, :] = Wlin_ref[:, :].T
        prow = jax.lax.broadcasted_iota(jnp.int32, (_V, 12), 0)
        pcol = jax.lax.broadcasted_iota(jnp.int32, (_V, 12), 1)
        psel = (pcol // 3 == prow).astype(f32)
        P1_ref[0:64, :] = jnp.dot(We1_ref[:_V, :].T * _RS132, psel,
                                  preferred_element_type=f32)
        P1_ref[64:68, :] = psel
        P1_ref[68:72, :] = jnp.zeros((4, 12), f32)

        nvT = nv_full_ref[:, :].T
        A_s = jnp.dot(nf_full_ref[:, :], We1_ref[_V:_V + _F, :],
                      preferred_element_type=f32) * _RS132
        A_sT = A_s.T
        nvT_t_ref[:, :] = jnp.concatenate([nvT] * _BR, axis=1)
        A_sT_t_ref[:, :] = jnp.concatenate([A_sT] * _BR, axis=1)
        PE_ref[12:16, :] = jnp.zeros((4, _BR), f32)
        RD_ref[12:16, :] = jnp.zeros((4, _L), f32)

    r0 = i * _BR
    lane = jax.lax.broadcasted_iota(jnp.int32, (1, _L), 1)
    s_id = lane % _N
    r_id = r0 + lane // _N
    mask = (s_id != r_id).astype(f32)

    R = R_ref[:, :]
    RT = RT_ref[:, :]

    nf_blk = nf_ref[:, :]
    nv_blk = nv_ref[:, :]
    PE_ref[0:12, :] = nv_blk.T
    PE_ref[16:80, :] = jnp.dot(We1rT_ref[:, :], nf_blk.T,
                               preferred_element_type=f32)
    exp = jnp.dot(PE_ref[:, :], R, preferred_element_type=f32)
    nv_r_exp = exp[0:12, :]
    A_r_exp = exp[16:80, :]

    diff = nv_r_exp - nvT_t_ref[:, :]
    hc = jnp.dot(P1_ref[:, :], diff * diff,
                 preferred_element_type=f32)
    n2 = hc[64:68, :]
    length = jnp.sqrt(jnp.maximum(n2, 1e-20))

    h1 = _silu(hc[0:64, :] + A_sT_t_ref[:, :] + A_r_exp)
    m = _silu(jnp.dot(We2T_ref[:, :], h1,
                      preferred_element_type=f32))
    p = _silu(jnp.dot(Wx1T_ref[:, :], m,
                      preferred_element_type=f32))
    p = _silu(jnp.dot(Wx2T_ref[:, :], p,
                      preferred_element_type=f32))
    phi = jnp.dot(WlinT_ref[:, :], p,
                  preferred_element_type=f32) + blin_ref[:, :]

    g = phi / (1.0 + length) * mask
    grow = jax.lax.broadcasted_iota(jnp.int32, (12, _V), 0)
    gcol = jax.lax.broadcasted_iota(jnp.int32, (12, _V), 1)
    sel12 = (grow // 3 == gcol).astype(f32)
    g12 = jnp.dot(sel12, g, preferred_element_type=f32)
    e = _sigmoid(jnp.dot(WinfT_ref[:, :], m,
                         preferred_element_type=f32))
    RD_ref[0:12, :] = g12 * diff
    RD_ref[16:80, :] = m * (e * mask)
    red = jnp.dot(RD_ref[:, :], RT, preferred_element_type=f32)
    shifts = red[0:12, :]
    m_i = red[16:80, :]

    m_i_n = m_i.T
    hh = jnp.dot(m_i_n, Wh1_ref[:_H, :], preferred_element_type=f32)
    hh = hh + jnp.dot(nf_blk, Wh1_ref[_H:, :], preferred_element_type=f32)
    hh = _silu(hh * _RS128)
    hh = _silu(jnp.dot(hh, Wh2_ref[:, :],
                       preferred_element_type=f32) * _RS64)
    fout_ref[:, :] = (jnp.dot(hh, Wout_ref[:, :],
                              preferred_element_type=f32) * _RS64 + nf_blk)

    vout_ref[:, :] = nv_blk + shifts.T * _INV_DEG


@jax.jit
def kernel(node_vectors, node_features, We1, We2, Wx1, Wx2, Winf, Wlin,
           blin, Wh1, Wh2, Wout):
    f32 = jnp.float32
    nv = node_vectors.reshape(_N, _V * 3).astype(f32)
    nf = node_features.astype(f32)

    seg = np.arange(_L) // _N
    R = jnp.asarray(seg[None, :] == np.arange(_BR)[:, None], dtype=f32)
    RT = jnp.asarray(seg[:, None] == np.arange(_BR)[None, :], dtype=f32)

    grid = (_N // _BR,)

    def full(shape):
        nd = len(shape)
        return pl.BlockSpec(shape, lambda i: (0,) * nd)

    out_shape = [
        jax.ShapeDtypeStruct((_N, 12), f32),
        jax.ShapeDtypeStruct((_N, _F), f32),
    ]
    out_specs = [
        pl.BlockSpec((_BR, 12), lambda i: (i, 0)),
        pl.BlockSpec((_BR, _F), lambda i: (i, 0)),
    ]
    in_arrays = [
        nv, nf, nv, nf, R, RT,
        We1, We2, Wx1, Wx2, Winf, Wlin, blin.reshape(_V, 1),
        Wh1, Wh2, Wout,
    ]
    in_specs = [full(a.shape) for a in in_arrays]
    in_specs[0] = pl.BlockSpec((_BR, 12), lambda i: (i, 0))
    in_specs[1] = pl.BlockSpec((_BR, _F), lambda i: (i, 0))

    vout, fout = pl.pallas_call(
        _egcl_body,
        grid=grid,
        in_specs=in_specs,
        out_specs=out_specs,
        out_shape=out_shape,
        scratch_shapes=[
            pltpu.VMEM((12, _L), f32),
            pltpu.VMEM((64, _L), f32),
            pltpu.VMEM((72, 12), f32),
            pltpu.VMEM((64, 64), f32),
            pltpu.VMEM((64, 64), f32),
            pltpu.VMEM((64, 64), f32),
            pltpu.VMEM((64, 64), f32),
            pltpu.VMEM((1, 64), f32),
            pltpu.VMEM((_V, 64), f32),
            pltpu.VMEM((80, _BR), f32),
            pltpu.VMEM((80, _L), f32),
        ],
    )(*in_arrays)

    return vout.reshape(_N, _V, 3), fout
